# Initial kernel scaffold; baseline (speedup 1.0000x reference)
import jax
import jax.numpy as jnp
from jax.experimental import pallas as pl


def kernel(x_s, x_t, idx, contrast_idx, W_cls_s, b_cls_s, W_cls_t, b_cls_t, W_clu_s, b_clu_s, W_clu_t, b_clu_t, memory_v1, memory_v2, memory_c1, memory_c2):
    raise NotImplementedError("write your pallas kernel here")


# speedup vs baseline: 4.3139x; 4.3139x over previous
import functools

import jax
import jax.numpy as jnp
from jax import lax
from jax.experimental import pallas as pl
from jax.experimental.pallas import tpu as pltpu
from jax.experimental.pallas import tpu_sc as plsc

EPS = 1e-07
N_DATA = 100000
NCE_K = 256
NCE_T = 0.07
KP1 = NCE_K + 1
B = 1024
DPAD = 272
D0 = 256

NC = 2
NS = 16
L = 16
NW = NC * NS
BPW = B // NW
NCHUNK = 64
NCH = NCE_K // NCHUNK


def _embed_body(x_ref, w_ref, b_ref, y_ref):
    y = lax.dot_general(x_ref[...], w_ref[...], (((1,), (0,)), ((), ())),
                        precision=lax.Precision.HIGHEST,
                        preferred_element_type=jnp.float32)
    y = y + b_ref[...]
    n = jnp.sqrt(jnp.sum(y * y, axis=1, keepdims=True))
    y_ref[...] = y / n


def _embed(x, W, b):
    Wp = jnp.pad(W, ((0, DPAD - KP1), (0, 0))).T
    bp = jnp.pad(b, (0, DPAD - KP1)).reshape(1, DPAD)
    return pl.pallas_call(
        _embed_body,
        out_shape=jax.ShapeDtypeStruct((B, DPAD), jnp.float32),
    )(x, Wp, bp)


_GATHER_DNUMS = lax.GatherDimensionNumbers(
    offset_dims=(), collapsed_slice_dims=(0,), start_index_map=(0,))


def _shuf(v, idx):
    return lax.gather(v, idx[:, None], _GATHER_DNUMS, (1,),
                      mode=lax.GatherScatterMode.PROMISE_IN_BOUNDS)


def _dot_row(buf, r, ychunks, perms):
    acc = buf[r, pl.ds(0, L)] * ychunks[0]
    for i in range(1, 16):
        acc = acc + buf[r, pl.ds(i * L, L)] * ychunks[i]
    for p in perms:
        acc = acc + _shuf(acc, p)
    return acc


def _sc_body(mem1, mem2, t1, t2, idxp, idxn, ys, yt,
             o1n, o2n, ot1n, ot2n, o1p, o2p, ot1p, ot2p,
             ip_v, in_v, ys_v, yt_v, p1_v, p2_v, n1_v, n2_v,
             nt1_v, nt2_v, pt1_v, pt2_v, o1_v, o2_v, sem, sem2):
    w = lax.axis_index("s") * NC + lax.axis_index("c")
    base = w * BPW

    pltpu.sync_copy(idxp.at[pl.ds(base, BPW)], ip_v)
    pltpu.sync_copy(idxn.at[pl.ds(base, BPW)], in_v)
    pltpu.sync_copy(ys.at[pl.ds(base, BPW)], ys_v)
    pltpu.sync_copy(yt.at[pl.ds(base, BPW)], yt_v)

    tp1 = pltpu.async_copy(t1.at[ip_v], pt1_v, sem2)
    tp2 = pltpu.async_copy(t2.at[ip_v], pt2_v, sem2)

    g1 = pltpu.async_copy(mem1.at[ip_v, pl.ds(0, D0)], p1_v, sem)
    g2 = pltpu.async_copy(mem2.at[ip_v, pl.ds(0, D0)], p2_v, sem)
    g1.wait()
    g2.wait()

    lane = lax.iota(jnp.int32, L)
    zero16 = jnp.zeros((L,), jnp.float32)
    perms = [(lane + s) % L for s in (8, 4, 2, 1)]

    def _y_chunks(y_v, bl):
        return [y_v[bl, pl.ds(i * L, L)] for i in range(16)]

    def b_body(bl, _):
        tn = []
        for h in range(2):
            tn.append(pltpu.async_copy(t1.at[in_v.at[bl, pl.ds(h * 128, 128)]],
                                       nt1_v.at[bl, pl.ds(h * 128, 128)], sem2))
            tn.append(pltpu.async_copy(t2.at[in_v.at[bl, pl.ds(h * 128, 128)]],
                                       nt2_v.at[bl, pl.ds(h * 128, 128)], sem2))

        ysc = _y_chunks(ys_v, bl)
        ytc = _y_chunks(yt_v, bl)

        def c_body(c, _):
            gg1 = pltpu.async_copy(
                mem1.at[in_v.at[bl, pl.ds(c * NCHUNK, NCHUNK)], pl.ds(0, D0)],
                n1_v, sem)
            gg2 = pltpu.async_copy(
                mem2.at[in_v.at[bl, pl.ds(c * NCHUNK, NCHUNK)], pl.ds(0, D0)],
                n2_v, sem)
            gg1.wait()
            gg2.wait()

            def g_body(g, _):
                res1 = zero16
                res2 = zero16
                for j in range(L):
                    r = g * L + j
                    res1 = jnp.where(lane == j, _dot_row(n2_v, r, ysc, perms), res1)
                    res2 = jnp.where(lane == j, _dot_row(n1_v, r, ytc, perms), res2)
                col = c * NCHUNK + g * L
                o1_v[bl, pl.ds(col, L)] = res1
                o2_v[bl, pl.ds(col, L)] = res2
                return 0

            lax.fori_loop(0, NCHUNK // L, g_body, 0)
            return 0

        lax.fori_loop(0, NCH, c_body, 0)
        for t in tn:
            t.wait()
        return 0

    lax.fori_loop(0, BPW, b_body, 0)

    tp1.wait()
    tp2.wait()

    def p_body(g, _):
        res1 = zero16
        res2 = zero16
        for j in range(L):
            bl = g * L + j
            ysc = _y_chunks(ys_v, bl)
            ytc = _y_chunks(yt_v, bl)
            res1 = jnp.where(lane == j, _dot_row(p2_v, bl, ysc, perms), res1)
            res2 = jnp.where(lane == j, _dot_row(p1_v, bl, ytc, perms), res2)
        p1_v[0, pl.ds(g * L, L)] = res1
        p2_v[0, pl.ds(g * L, L)] = res2
        return 0

    lax.fori_loop(0, BPW // L, p_body, 0)

    pltpu.sync_copy(o1_v, o1n.at[pl.ds(base, BPW)])
    pltpu.sync_copy(o2_v, o2n.at[pl.ds(base, BPW)])
    pltpu.sync_copy(nt1_v, ot1n.at[pl.ds(base, BPW)])
    pltpu.sync_copy(nt2_v, ot2n.at[pl.ds(base, BPW)])
    pltpu.sync_copy(p1_v.at[0, pl.ds(0, BPW)], o1p.at[pl.ds(base, BPW)])
    pltpu.sync_copy(p2_v.at[0, pl.ds(0, BPW)], o2p.at[pl.ds(base, BPW)])
    pltpu.sync_copy(pt1_v, ot1p.at[pl.ds(base, BPW)])
    pltpu.sync_copy(pt2_v, ot2p.at[pl.ds(base, BPW)])


_sc_gather_dot = functools.partial(
    pl.kernel,
    mesh=plsc.VectorSubcoreMesh(core_axis_name="c", subcore_axis_name="s"),
    out_type=[jax.ShapeDtypeStruct((B, NCE_K), jnp.float32),
              jax.ShapeDtypeStruct((B, NCE_K), jnp.float32),
              jax.ShapeDtypeStruct((B, NCE_K), jnp.float32),
              jax.ShapeDtypeStruct((B, NCE_K), jnp.float32),
              jax.ShapeDtypeStruct((B,), jnp.float32),
              jax.ShapeDtypeStruct((B,), jnp.float32),
              jax.ShapeDtypeStruct((B,), jnp.float32),
              jax.ShapeDtypeStruct((B,), jnp.float32)],
    scratch_types=[
        pltpu.VMEM((BPW,), jnp.int32),
        pltpu.VMEM((BPW, NCE_K), jnp.int32),
        pltpu.VMEM((BPW, D0), jnp.float32),
        pltpu.VMEM((BPW, D0), jnp.float32),
        pltpu.VMEM((BPW, D0), jnp.float32),
        pltpu.VMEM((BPW, D0), jnp.float32),
        pltpu.VMEM((NCHUNK, D0), jnp.float32),
        pltpu.VMEM((NCHUNK, D0), jnp.float32),
        pltpu.VMEM((BPW, NCE_K), jnp.float32),
        pltpu.VMEM((BPW, NCE_K), jnp.float32),
        pltpu.VMEM((BPW,), jnp.float32),
        pltpu.VMEM((BPW,), jnp.float32),
        pltpu.VMEM((BPW, NCE_K), jnp.float32),
        pltpu.VMEM((BPW, NCE_K), jnp.float32),
        pltpu.SemaphoreType.DMA,
        pltpu.SemaphoreType.DMA,
    ],
)(_sc_body)


def _loss_body(s1n_ref, s2n_ref, t1n_ref, t2n_ref,
               s1p_ref, s2p_ref, t1p_ref, t2p_ref,
               yst_ref, ytt_ref, out_ref):
    c = float(NCE_K) / float(N_DATA)

    def one(sn, sp):
        en = jnp.exp(sn * (1.0 / NCE_T))
        ep = jnp.exp(sp * (1.0 / NCE_T))
        Z = (jnp.sum(en) + jnp.sum(ep)) * (float(N_DATA) / (B * KP1))
        lD1 = jnp.log((ep / Z) / (ep / Z + (c + EPS)))
        lD0 = jnp.log(c / (en / Z + (c + EPS)))
        return -(jnp.sum(lD1) + jnp.sum(lD0)) / B

    yst = yst_ref[...]
    ytt = ytt_ref[...]
    s1n = s1n_ref[...] + t2n_ref[...] * yst
    s2n = s2n_ref[...] + t1n_ref[...] * ytt
    s1p = s1p_ref[...] + t2p_ref[...] * yst
    s2p = s2p_ref[...] + t1p_ref[...] * ytt
    out_ref[...] = jnp.reshape(one(s1n, s1p) + one(s2n, s2p), (1, 1))


def _loss(s1n, s2n, t1n, t2n, s1p, s2p, t1p, t2p, yst, ytt):
    out = pl.pallas_call(
        _loss_body,
        out_shape=jax.ShapeDtypeStruct((1, 1), jnp.float32),
    )(s1n, s2n, t1n, t2n,
      s1p.reshape(B, 1), s2p.reshape(B, 1),
      t1p.reshape(B, 1), t2p.reshape(B, 1),
      yst.reshape(B, 1), ytt.reshape(B, 1))
    return out.reshape(1)


def kernel(x_s, x_t, idx, contrast_idx, W_cls_s, b_cls_s, W_cls_t, b_cls_t,
           W_clu_s, b_clu_s, W_clu_t, b_clu_t,
           memory_v1, memory_v2, memory_c1, memory_c2):
    y_s = _embed(x_s, W_clu_s, b_clu_s)
    y_t = _embed(x_t, W_clu_t, b_clu_t)
    t1 = memory_c1[:, NCE_K]
    t2 = memory_c2[:, NCE_K]
    s1n, s2n, t1n, t2n, s1p, s2p, t1p, t2p = _sc_gather_dot(
        memory_c1, memory_c2, t1, t2,
        idx.astype(jnp.int32), contrast_idx.astype(jnp.int32),
        y_s[:, :D0], y_t[:, :D0])
    return _loss(s1n, s2n, t1n, t2n, s1p, s2p, t1p, t2p,
                 y_s[:, NCE_K], y_t[:, NCE_K])


# speedup vs baseline: 6.2735x; 1.4543x over previous
import functools

import jax
import jax.numpy as jnp
from jax import lax
from jax.experimental import pallas as pl
from jax.experimental.pallas import tpu as pltpu
from jax.experimental.pallas import tpu_sc as plsc

EPS = 1e-07
N_DATA = 100000
NCE_K = 256
NCE_T = 0.07
KP1 = NCE_K + 1
B = 1024
DPAD = 272
D0 = 256

NC = 2
NS = 16
L = 16
NW = NC * NS
BPW = B // NW
NCHUNK = 32
NCH = NCE_K // NCHUNK
NSTEP = BPW * NCH


def _embed_body(x_ref, w_ref, b_ref, y_ref):
    y = lax.dot_general(x_ref[...], w_ref[...], (((1,), (0,)), ((), ())),
                        precision=lax.Precision.HIGHEST,
                        preferred_element_type=jnp.float32)
    y = y + b_ref[...]
    n = jnp.sqrt(jnp.sum(y * y, axis=1, keepdims=True))
    y_ref[...] = y / n


def _embed(x, W, b):
    Wp = jnp.pad(W, ((0, DPAD - KP1), (0, 0))).T
    bp = jnp.pad(b, (0, DPAD - KP1)).reshape(1, DPAD)
    return pl.pallas_call(
        _embed_body,
        out_shape=jax.ShapeDtypeStruct((B, DPAD), jnp.float32),
    )(x, Wp, bp)


_GATHER_DNUMS = lax.GatherDimensionNumbers(
    offset_dims=(), collapsed_slice_dims=(0,), start_index_map=(0,))


def _shuf(v, idx):
    return lax.gather(v, idx[:, None], _GATHER_DNUMS, (1,),
                      mode=lax.GatherScatterMode.PROMISE_IN_BOUNDS)


def _dot_row(buf, r, ychunks, perms):
    acc = buf[r, pl.ds(0, L)] * ychunks[0]
    for i in range(1, 16):
        acc = acc + buf[r, pl.ds(i * L, L)] * ychunks[i]
    for p in perms:
        acc = acc + _shuf(acc, p)
    return acc


def _dot_row2(buf, par, r, ychunks, perms):
    acc = buf[par, r, pl.ds(0, L)] * ychunks[0]
    for i in range(1, 16):
        acc = acc + buf[par, r, pl.ds(i * L, L)] * ychunks[i]
    for p in perms:
        acc = acc + _shuf(acc, p)
    return acc


def _sc_body(mem1, mem2, t1, t2, idxp, idxn, ys, yt,
             o1n, o2n, ot1n, ot2n, o1p, o2p, ot1p, ot2p,
             ip_v, in_v, ys_v, yt_v, p1_v, p2_v, n1_v, n2_v,
             nt1_v, nt2_v, pt1_v, pt2_v, o1_v, o2_v, sem, sem2):
    w = lax.axis_index("s") * NC + lax.axis_index("c")
    base = w * BPW

    pltpu.sync_copy(idxp.at[pl.ds(base, BPW)], ip_v)
    pltpu.sync_copy(idxn.at[pl.ds(base, BPW)], in_v)
    pltpu.sync_copy(ys.at[pl.ds(base, BPW)], ys_v)
    pltpu.sync_copy(yt.at[pl.ds(base, BPW)], yt_v)

    tp1 = pltpu.async_copy(t1.at[ip_v], pt1_v, sem2)
    tp2 = pltpu.async_copy(t2.at[ip_v], pt2_v, sem2)

    g1 = pltpu.async_copy(mem1.at[ip_v, pl.ds(0, D0)], p1_v, sem)
    g2 = pltpu.async_copy(mem2.at[ip_v, pl.ds(0, D0)], p2_v, sem)
    g1.wait()
    g2.wait()

    lane = lax.iota(jnp.int32, L)
    zero16 = jnp.zeros((L,), jnp.float32)
    perms = [(lane + s) % L for s in (8, 4, 2, 1)]

    def _y_chunks(y_v, bl):
        return [y_v[bl, pl.ds(i * L, L)] for i in range(16)]

    def _fire(s):
        bl = s // NCH
        c = s - bl * NCH
        par = s % 2
        isl = in_v.at[bl, pl.ds(c * NCHUNK, NCHUNK)]
        pltpu.async_copy(mem1.at[isl, pl.ds(0, D0)], n1_v.at[par], sem)
        pltpu.async_copy(mem2.at[isl, pl.ds(0, D0)], n2_v.at[par], sem)

    _fire(0)

    def s_body(s, _):
        @pl.when(s + 1 < NSTEP)
        def _():
            _fire(s + 1)

        pltpu.make_async_copy(mem1.at[pl.ds(0, NCHUNK), pl.ds(0, D0)],
                              n1_v.at[0], sem).wait()
        pltpu.make_async_copy(mem1.at[pl.ds(0, NCHUNK), pl.ds(0, D0)],
                              n2_v.at[0], sem).wait()

        bl = s // NCH
        c = s - bl * NCH
        par = s % 2

        @pl.when(c == 0)
        def _():
            for h in range(2):
                pltpu.async_copy(t1.at[in_v.at[bl, pl.ds(h * 128, 128)]],
                                 nt1_v.at[bl, pl.ds(h * 128, 128)], sem2)
                pltpu.async_copy(t2.at[in_v.at[bl, pl.ds(h * 128, 128)]],
                                 nt2_v.at[bl, pl.ds(h * 128, 128)], sem2)

        ysc = _y_chunks(ys_v, bl)
        ytc = _y_chunks(yt_v, bl)

        def g_body(g, _):
            res1 = zero16
            res2 = zero16
            for j in range(L):
                r = g * L + j
                res1 = jnp.where(lane == j, _dot_row2(n2_v, par, r, ysc, perms), res1)
                res2 = jnp.where(lane == j, _dot_row2(n1_v, par, r, ytc, perms), res2)
            col = c * NCHUNK + g * L
            o1_v[bl, pl.ds(col, L)] = res1
            o2_v[bl, pl.ds(col, L)] = res2
            return 0

        lax.fori_loop(0, NCHUNK // L, g_body, 0)
        return 0

    lax.fori_loop(0, NSTEP, s_body, 0)

    def td_body(bl, _):
        pltpu.make_async_copy(t1.at[pl.ds(0, NCE_K)], nt1_v.at[bl], sem2).wait()
        pltpu.make_async_copy(t1.at[pl.ds(0, NCE_K)], nt2_v.at[bl], sem2).wait()
        return 0

    lax.fori_loop(0, BPW, td_body, 0)

    tp1.wait()
    tp2.wait()

    def p_body(g, _):
        res1 = zero16
        res2 = zero16
        for j in range(L):
            bl = g * L + j
            ysc = _y_chunks(ys_v, bl)
            ytc = _y_chunks(yt_v, bl)
            res1 = jnp.where(lane == j, _dot_row(p2_v, bl, ysc, perms), res1)
            res2 = jnp.where(lane == j, _dot_row(p1_v, bl, ytc, perms), res2)
        p1_v[0, pl.ds(g * L, L)] = res1
        p2_v[0, pl.ds(g * L, L)] = res2
        return 0

    lax.fori_loop(0, BPW // L, p_body, 0)

    pltpu.sync_copy(o1_v, o1n.at[pl.ds(base, BPW)])
    pltpu.sync_copy(o2_v, o2n.at[pl.ds(base, BPW)])
    pltpu.sync_copy(nt1_v, ot1n.at[pl.ds(base, BPW)])
    pltpu.sync_copy(nt2_v, ot2n.at[pl.ds(base, BPW)])
    pltpu.sync_copy(p1_v.at[0, pl.ds(0, BPW)], o1p.at[pl.ds(base, BPW)])
    pltpu.sync_copy(p2_v.at[0, pl.ds(0, BPW)], o2p.at[pl.ds(base, BPW)])
    pltpu.sync_copy(pt1_v, ot1p.at[pl.ds(base, BPW)])
    pltpu.sync_copy(pt2_v, ot2p.at[pl.ds(base, BPW)])


_sc_gather_dot = functools.partial(
    pl.kernel,
    mesh=plsc.VectorSubcoreMesh(core_axis_name="c", subcore_axis_name="s"),
    out_type=[jax.ShapeDtypeStruct((B, NCE_K), jnp.float32),
              jax.ShapeDtypeStruct((B, NCE_K), jnp.float32),
              jax.ShapeDtypeStruct((B, NCE_K), jnp.float32),
              jax.ShapeDtypeStruct((B, NCE_K), jnp.float32),
              jax.ShapeDtypeStruct((B,), jnp.float32),
              jax.ShapeDtypeStruct((B,), jnp.float32),
              jax.ShapeDtypeStruct((B,), jnp.float32),
              jax.ShapeDtypeStruct((B,), jnp.float32)],
    scratch_types=[
        pltpu.VMEM((BPW,), jnp.int32),
        pltpu.VMEM((BPW, NCE_K), jnp.int32),
        pltpu.VMEM((BPW, D0), jnp.float32),
        pltpu.VMEM((BPW, D0), jnp.float32),
        pltpu.VMEM((BPW, D0), jnp.float32),
        pltpu.VMEM((BPW, D0), jnp.float32),
        pltpu.VMEM((2, NCHUNK, D0), jnp.float32),
        pltpu.VMEM((2, NCHUNK, D0), jnp.float32),
        pltpu.VMEM((BPW, NCE_K), jnp.float32),
        pltpu.VMEM((BPW, NCE_K), jnp.float32),
        pltpu.VMEM((BPW,), jnp.float32),
        pltpu.VMEM((BPW,), jnp.float32),
        pltpu.VMEM((BPW, NCE_K), jnp.float32),
        pltpu.VMEM((BPW, NCE_K), jnp.float32),
        pltpu.SemaphoreType.DMA,
        pltpu.SemaphoreType.DMA,
    ],
)(_sc_body)


def _loss_body(s1n_ref, s2n_ref, t1n_ref, t2n_ref,
               s1p_ref, s2p_ref, t1p_ref, t2p_ref,
               yst_ref, ytt_ref, out_ref):
    c = float(NCE_K) / float(N_DATA)

    def one(sn, sp):
        en = jnp.exp(sn * (1.0 / NCE_T))
        ep = jnp.exp(sp * (1.0 / NCE_T))
        Z = (jnp.sum(en) + jnp.sum(ep)) * (float(N_DATA) / (B * KP1))
        lD1 = jnp.log((ep / Z) / (ep / Z + (c + EPS)))
        lD0 = jnp.log(c / (en / Z + (c + EPS)))
        return -(jnp.sum(lD1) + jnp.sum(lD0)) / B

    yst = yst_ref[...]
    ytt = ytt_ref[...]
    s1n = s1n_ref[...] + t2n_ref[...] * yst
    s2n = s2n_ref[...] + t1n_ref[...] * ytt
    s1p = s1p_ref[...] + t2p_ref[...] * yst
    s2p = s2p_ref[...] + t1p_ref[...] * ytt
    out_ref[...] = jnp.reshape(one(s1n, s1p) + one(s2n, s2p), (1, 1))


def _loss(s1n, s2n, t1n, t2n, s1p, s2p, t1p, t2p, yst, ytt):
    out = pl.pallas_call(
        _loss_body,
        out_shape=jax.ShapeDtypeStruct((1, 1), jnp.float32),
    )(s1n, s2n, t1n, t2n,
      s1p.reshape(B, 1), s2p.reshape(B, 1),
      t1p.reshape(B, 1), t2p.reshape(B, 1),
      yst.reshape(B, 1), ytt.reshape(B, 1))
    return out.reshape(1)


def kernel(x_s, x_t, idx, contrast_idx, W_cls_s, b_cls_s, W_cls_t, b_cls_t,
           W_clu_s, b_clu_s, W_clu_t, b_clu_t,
           memory_v1, memory_v2, memory_c1, memory_c2):
    y_s = _embed(x_s, W_clu_s, b_clu_s)
    y_t = _embed(x_t, W_clu_t, b_clu_t)
    t1 = memory_c1[:, NCE_K]
    t2 = memory_c2[:, NCE_K]
    s1n, s2n, t1n, t2n, s1p, s2p, t1p, t2p = _sc_gather_dot(
        memory_c1, memory_c2, t1, t2,
        idx.astype(jnp.int32), contrast_idx.astype(jnp.int32),
        y_s[:, :D0], y_t[:, :D0])
    return _loss(s1n, s2n, t1n, t2n, s1p, s2p, t1p, t2p,
                 y_s[:, NCE_K], y_t[:, NCE_K])


# speedup vs baseline: 8.4680x; 1.3498x over previous
import functools

import jax
import jax.numpy as jnp
from jax import lax
from jax.experimental import pallas as pl
from jax.experimental.pallas import tpu as pltpu
from jax.experimental.pallas import tpu_sc as plsc

EPS = 1e-07
N_DATA = 100000
NCE_K = 256
NCE_T = 0.07
KP1 = NCE_K + 1
B = 1024
DPAD = 272
D0 = 256

NC = 2
NS = 16
L = 16
NW = NC * NS
BPW = B // NW
NCHUNK = 32
NCH = NCE_K // NCHUNK
NBUF = 4
NSTEP = BPW * NCH


def _embed_body(x_ref, w_ref, b_ref, y_ref):
    y = lax.dot_general(x_ref[...], w_ref[...], (((1,), (0,)), ((), ())),
                        precision=lax.Precision.HIGHEST,
                        preferred_element_type=jnp.float32)
    y = y + b_ref[...]
    n = jnp.sqrt(jnp.sum(y * y, axis=1, keepdims=True))
    y_ref[...] = y / n


def _embed(x, W, b):
    Wp = jnp.pad(W, ((0, DPAD - KP1), (0, 0))).T
    bp = jnp.pad(b, (0, DPAD - KP1)).reshape(1, DPAD)
    return pl.pallas_call(
        _embed_body,
        out_shape=jax.ShapeDtypeStruct((B, DPAD), jnp.float32),
    )(x, Wp, bp)


_GATHER_DNUMS = lax.GatherDimensionNumbers(
    offset_dims=(), collapsed_slice_dims=(0,), start_index_map=(0,))


def _shuf(v, idx):
    return lax.gather(v, idx[:, None], _GATHER_DNUMS, (1,),
                      mode=lax.GatherScatterMode.PROMISE_IN_BOUNDS)


def _dot_rows2(buf, r, ychunks, perms):
    acc = buf[r, pl.ds(0, L)] * ychunks[0]
    for i in range(1, 16):
        acc = acc + buf[r, pl.ds(i * L, L)] * ychunks[i]
    for p in perms:
        acc = acc + _shuf(acc, p)
    return acc


def _dot_rows3(buf, par, r, ychunks, perms):
    acc = buf[par, r, pl.ds(0, L)] * ychunks[0]
    for i in range(1, 16):
        acc = acc + buf[par, r, pl.ds(i * L, L)] * ychunks[i]
    for p in perms:
        acc = acc + _shuf(acc, p)
    return acc


def _sc_body(mem, tl, idxp, idxn, y,
             on, op, otn, otp,
             ip_v, in_v, y_v, p_v, n_v, nt_v, pt_v, o_v, sem, sem2):
    w = lax.axis_index("s") * NC + lax.axis_index("c")
    base = w * BPW

    pltpu.sync_copy(idxp.at[pl.ds(base, BPW)], ip_v)
    pltpu.sync_copy(idxn.at[pl.ds(base, BPW)], in_v)
    pltpu.sync_copy(y.at[pl.ds(base, BPW)], y_v)

    tp = pltpu.async_copy(tl.at[ip_v], pt_v, sem2)

    gp = pltpu.async_copy(mem.at[ip_v, pl.ds(0, D0)], p_v, sem)
    gp.wait()

    lane = lax.iota(jnp.int32, L)
    zero16 = jnp.zeros((L,), jnp.float32)
    perms = [(lane + s) % L for s in (8, 4, 2, 1)]

    def _y_chunks(bl):
        return [y_v[bl, pl.ds(i * L, L)] for i in range(16)]

    def _fire(s):
        bl = s // NCH
        c = s - bl * NCH
        par = s % NBUF
        isl = in_v.at[bl, pl.ds(c * NCHUNK, NCHUNK)]
        pltpu.async_copy(mem.at[isl, pl.ds(0, D0)], n_v.at[par], sem)

    for s0 in range(NBUF - 1):
        _fire(s0)

    def s_body(s, _):
        @pl.when(s + NBUF - 1 < NSTEP)
        def _():
            _fire(s + NBUF - 1)

        pltpu.make_async_copy(mem.at[pl.ds(0, NCHUNK), pl.ds(0, D0)],
                              n_v.at[0], sem).wait()

        bl = s // NCH
        c = s - bl * NCH
        par = s % NBUF

        @pl.when(c == 0)
        def _():
            for h in range(2):
                pltpu.async_copy(tl.at[in_v.at[bl, pl.ds(h * 128, 128)]],
                                 nt_v.at[bl, pl.ds(h * 128, 128)], sem2)

        ysc = _y_chunks(bl)

        def g_body(g, _):
            res = zero16
            for j in range(L):
                r = g * L + j
                res = jnp.where(lane == j, _dot_rows3(n_v, par, r, ysc, perms), res)
            o_v[bl, pl.ds(c * NCHUNK + g * L, L)] = res
            return 0

        lax.fori_loop(0, NCHUNK // L, g_body, 0)
        return 0

    lax.fori_loop(0, NSTEP, s_body, 0)

    def td_body(bl, _):
        pltpu.make_async_copy(tl.at[pl.ds(0, NCE_K)], nt_v.at[bl], sem2).wait()
        return 0

    lax.fori_loop(0, BPW, td_body, 0)
    tp.wait()

    def p_body(g, _):
        res = zero16
        for j in range(L):
            bl = g * L + j
            res = jnp.where(lane == j, _dot_rows2(p_v, bl, _y_chunks(bl), perms), res)
        p_v[0, pl.ds(g * L, L)] = res
        return 0

    lax.fori_loop(0, BPW // L, p_body, 0)

    pltpu.sync_copy(o_v, on.at[pl.ds(base, BPW)])
    pltpu.sync_copy(nt_v, otn.at[pl.ds(base, BPW)])
    pltpu.sync_copy(p_v.at[0, pl.ds(0, BPW)], op.at[pl.ds(base, BPW)])
    pltpu.sync_copy(pt_v, otp.at[pl.ds(base, BPW)])


_sc_bank = functools.partial(
    pl.kernel,
    mesh=plsc.VectorSubcoreMesh(core_axis_name="c", subcore_axis_name="s"),
    out_type=[jax.ShapeDtypeStruct((B, NCE_K), jnp.float32),
              jax.ShapeDtypeStruct((B,), jnp.float32),
              jax.ShapeDtypeStruct((B, NCE_K), jnp.float32),
              jax.ShapeDtypeStruct((B,), jnp.float32)],
    scratch_types=[
        pltpu.VMEM((BPW,), jnp.int32),
        pltpu.VMEM((BPW, NCE_K), jnp.int32),
        pltpu.VMEM((BPW, D0), jnp.float32),
        pltpu.VMEM((BPW, D0), jnp.float32),
        pltpu.VMEM((NBUF, NCHUNK, D0), jnp.float32),
        pltpu.VMEM((BPW, NCE_K), jnp.float32),
        pltpu.VMEM((BPW,), jnp.float32),
        pltpu.VMEM((BPW, NCE_K), jnp.float32),
        pltpu.SemaphoreType.DMA,
        pltpu.SemaphoreType.DMA,
    ],
)(_sc_body)


def _loss_body(s1n_ref, s2n_ref, t1n_ref, t2n_ref,
               s1p_ref, s2p_ref, t1p_ref, t2p_ref,
               yst_ref, ytt_ref, out_ref):
    c = float(NCE_K) / float(N_DATA)

    def one(sn, sp):
        en = jnp.exp(sn * (1.0 / NCE_T))
        ep = jnp.exp(sp * (1.0 / NCE_T))
        Z = (jnp.sum(en) + jnp.sum(ep)) * (float(N_DATA) / (B * KP1))
        lD1 = jnp.log((ep / Z) / (ep / Z + (c + EPS)))
        lD0 = jnp.log(c / (en / Z + (c + EPS)))
        return -(jnp.sum(lD1) + jnp.sum(lD0)) / B

    yst = yst_ref[...]
    ytt = ytt_ref[...]
    s1n = s1n_ref[...] + t2n_ref[...] * yst
    s2n = s2n_ref[...] + t1n_ref[...] * ytt
    s1p = s1p_ref[...] + t2p_ref[...] * yst
    s2p = s2p_ref[...] + t1p_ref[...] * ytt
    out_ref[...] = jnp.reshape(one(s1n, s1p) + one(s2n, s2p), (1, 1))


def _loss(s1n, s2n, t1n, t2n, s1p, s2p, t1p, t2p, yst, ytt):
    out = pl.pallas_call(
        _loss_body,
        out_shape=jax.ShapeDtypeStruct((1, 1), jnp.float32),
    )(s1n, s2n, t1n, t2n,
      s1p.reshape(B, 1), s2p.reshape(B, 1),
      t1p.reshape(B, 1), t2p.reshape(B, 1),
      yst.reshape(B, 1), ytt.reshape(B, 1))
    return out.reshape(1)


def kernel(x_s, x_t, idx, contrast_idx, W_cls_s, b_cls_s, W_cls_t, b_cls_t,
           W_clu_s, b_clu_s, W_clu_t, b_clu_t,
           memory_v1, memory_v2, memory_c1, memory_c2):
    y_s = _embed(x_s, W_clu_s, b_clu_s)
    y_t = _embed(x_t, W_clu_t, b_clu_t)
    t1 = memory_c1[:, NCE_K]
    t2 = memory_c2[:, NCE_K]
    idxi = idx.astype(jnp.int32)
    cidxi = contrast_idx.astype(jnp.int32)
    s2n, s2p, t1n, t1p = _sc_bank(memory_c1, t1, idxi, cidxi, y_t[:, :D0])
    s1n, s1p, t2n, t2p = _sc_bank(memory_c2, t2, idxi, cidxi, y_s[:, :D0])
    return _loss(s1n, s2n, t1n, t2n, s1p, s2p, t1p, t2p,
                 y_s[:, NCE_K], y_t[:, NCE_K])
